# baseline (speedup 1.0000x reference)
import functools

import jax
import jax.numpy as jnp
from jax import lax
from jax.experimental import pallas as pl
from jax.experimental.pallas import tpu as pltpu
from jax.experimental.pallas import tpu_sc as plsc

N = 10000
E = 320000
D = 128
N_PAD = 10240
CHUNK = 80
MM_BLOCK = 1024


def _mm_kernel(nf_ref, w1_ref, w2_ref, a_ref, b_ref):
    x = nf_ref[...]
    a_ref[...] = jnp.dot(x, w1_ref[...], preferred_element_type=jnp.float32)
    b_ref[...] = jnp.dot(x, w2_ref[...], preferred_element_type=jnp.float32)


def _precompute_tables(nf_pad, w1, w2):
    grid = N_PAD // MM_BLOCK
    out_sds = jax.ShapeDtypeStruct((N_PAD, D), jnp.float32)
    return pl.pallas_call(
        _mm_kernel,
        grid=(grid,),
        in_specs=[
            pl.BlockSpec((MM_BLOCK, D), lambda i: (i, 0)),
            pl.BlockSpec((D, D), lambda i: (0, 0)),
            pl.BlockSpec((D, D), lambda i: (0, 0)),
        ],
        out_specs=[
            pl.BlockSpec((MM_BLOCK, D), lambda i: (i, 0)),
            pl.BlockSpec((MM_BLOCK, D), lambda i: (i, 0)),
        ],
        out_shape=[out_sds, out_sds],
    )(nf_pad, w1, w2)


def _make_edge_kernel(num_workers, epw):
    mesh = plsc.VectorSubcoreMesh(core_axis_name="c", subcore_axis_name="s")
    n_chunks = epw // CHUNK

    @functools.partial(
        pl.kernel,
        mesh=mesh,
        out_type=jax.ShapeDtypeStruct((E, D), jnp.float32),
        scratch_types=[
            pltpu.VMEM((epw,), jnp.int32),
            pltpu.VMEM((epw,), jnp.int32),
            pltpu.VMEM((epw,), jnp.int32),
            pltpu.VMEM((CHUNK,), jnp.int32),
            pltpu.VMEM((CHUNK,), jnp.int32),
            pltpu.VMEM((CHUNK, D), jnp.float32),
            pltpu.VMEM((CHUNK, D), jnp.float32),
            pltpu.SemaphoreType.DMA,
            pltpu.SemaphoreType.DMA,
        ],
    )
    def edge_kernel(a_hbm, b_hbm, src_hbm, dst_hbm, conn_hbm, out_hbm,
                    src_v, dst_v, conn_v, msrc_v, mdst_v, arows_v, brows_v,
                    sem_a, sem_b):
        nc = num_workers // 16
        wid = lax.axis_index("s") * nc + lax.axis_index("c")
        base = wid * epw
        pltpu.sync_copy(src_hbm.at[pl.ds(base, epw)], src_v)
        pltpu.sync_copy(dst_hbm.at[pl.ds(base, epw)], dst_v)
        pltpu.sync_copy(conn_hbm.at[pl.ds(base, epw)], conn_v)

        def chunk_body(c, carry):
            off = c * CHUNK

            def mask_body(j, carry2):
                sl = pl.ds(off + j * 16, 16)
                s = src_v[sl]
                dd = dst_v[sl]
                cn = conn_v[sl]
                m = (cn == 1) & (s < dd)
                msrc_v[pl.ds(j * 16, 16)] = jnp.where(m, s, N)
                mdst_v[pl.ds(j * 16, 16)] = jnp.where(m, dd, N)
                return carry2

            lax.fori_loop(0, CHUNK // 16, mask_body, 0, unroll=True)

            ha = pltpu.async_copy(a_hbm.at[msrc_v], arows_v, sem_a)
            hb = pltpu.async_copy(b_hbm.at[mdst_v], brows_v, sem_b)
            ha.wait()
            hb.wait()

            def row_body(i, carry2):
                for j in range(D // 16):
                    sl = pl.ds(j * 16, 16)
                    v = arows_v[i, sl] + brows_v[i, sl]
                    arows_v[i, sl] = jnp.maximum(v, v * 0.01)
                return carry2

            lax.fori_loop(0, CHUNK, row_body, 0)

            pltpu.sync_copy(arows_v, out_hbm.at[pl.ds(base + off, CHUNK)])
            return carry

        lax.fori_loop(0, n_chunks, chunk_body, 0)

    return edge_kernel


def kernel(nf, edge_index, connected, W):
    ei = edge_index.astype(jnp.int32)
    src = ei[0]
    dst = ei[1]
    conn = connected.astype(jnp.int32)
    nf_pad = jnp.zeros((N_PAD, D), jnp.float32).at[:N, :].set(nf)
    w1 = W[:D, :]
    w2 = W[D:, :]
    a, b = _precompute_tables(nf_pad, w1, w2)

    info = plsc.get_sparse_core_info()
    num_workers = info.num_cores * info.num_subcores
    epw = E // num_workers
    edge_fn = _make_edge_kernel(num_workers, epw)
    return edge_fn(a, b, src, dst, conn)


# speedup vs baseline: 1.0008x; 1.0008x over previous
import functools

import jax
import jax.numpy as jnp
from jax import lax
from jax.experimental import pallas as pl
from jax.experimental.pallas import tpu as pltpu
from jax.experimental.pallas import tpu_sc as plsc

N = 10000
E = 320000
D = 128
N_PAD = 10240
CHUNK = 80
MM_BLOCK = 1024


def _mm_kernel(nf_ref, w1_ref, w2_ref, a_ref, b_ref):
    x = nf_ref[...]
    a_ref[...] = jnp.dot(x, w1_ref[...], preferred_element_type=jnp.float32)
    b_ref[...] = jnp.dot(x, w2_ref[...], preferred_element_type=jnp.float32)


def _precompute_tables(nf_pad, w1, w2):
    grid = N_PAD // MM_BLOCK
    out_sds = jax.ShapeDtypeStruct((N_PAD, D), jnp.float32)
    return pl.pallas_call(
        _mm_kernel,
        grid=(grid,),
        in_specs=[
            pl.BlockSpec((MM_BLOCK, D), lambda i: (i, 0)),
            pl.BlockSpec((D, D), lambda i: (0, 0)),
            pl.BlockSpec((D, D), lambda i: (0, 0)),
        ],
        out_specs=[
            pl.BlockSpec((MM_BLOCK, D), lambda i: (i, 0)),
            pl.BlockSpec((MM_BLOCK, D), lambda i: (i, 0)),
        ],
        out_shape=[out_sds, out_sds],
    )(nf_pad, w1, w2)


def _make_edge_kernel(num_workers, epw):
    mesh = plsc.VectorSubcoreMesh(core_axis_name="c", subcore_axis_name="s")
    n_chunks = epw // CHUNK

    @functools.partial(
        pl.kernel,
        mesh=mesh,
        out_type=jax.ShapeDtypeStruct((E, D), jnp.float32),
        scratch_types=[
            pltpu.VMEM((epw,), jnp.int32),
            pltpu.VMEM((epw,), jnp.int32),
            pltpu.VMEM((epw,), jnp.int32),
            pltpu.VMEM((CHUNK, D), jnp.float32),
            pltpu.VMEM((CHUNK, D), jnp.float32),
            pltpu.VMEM((CHUNK, D), jnp.float32),
            pltpu.VMEM((CHUNK, D), jnp.float32),
            pltpu.VMEM((CHUNK, D), jnp.float32),
            pltpu.VMEM((CHUNK, D), jnp.float32),
            pltpu.SemaphoreType.DMA,
            pltpu.SemaphoreType.DMA,
            pltpu.SemaphoreType.DMA,
            pltpu.SemaphoreType.DMA,
            pltpu.SemaphoreType.DMA,
            pltpu.SemaphoreType.DMA,
        ],
    )
    def edge_kernel(a_hbm, b_hbm, src_hbm, dst_hbm, conn_hbm, out_hbm,
                    src_v, dst_v, conn_v,
                    ra0, ra1, rb0, rb1, ro0, ro1,
                    sa0, sa1, sb0, sb1, sw0, sw1):
        rows_a = (ra0, ra1)
        rows_b = (rb0, rb1)
        rows_o = (ro0, ro1)
        sem_a = (sa0, sa1)
        sem_b = (sb0, sb1)
        sem_w = (sw0, sw1)

        nc = num_workers // 16
        wid = lax.axis_index("s") * nc + lax.axis_index("c")
        base = wid * epw
        pltpu.sync_copy(src_hbm.at[pl.ds(base, epw)], src_v)
        pltpu.sync_copy(dst_hbm.at[pl.ds(base, epw)], dst_v)
        pltpu.sync_copy(conn_hbm.at[pl.ds(base, epw)], conn_v)

        def mask_body(j, carry):
            sl = pl.ds(j * 16, 16)
            s = src_v[sl]
            dd = dst_v[sl]
            m = (conn_v[sl] == 1) & (s < dd)
            src_v[sl] = jnp.where(m, s, N)
            dst_v[sl] = jnp.where(m, dd, N)
            return carry

        lax.fori_loop(0, epw // 16, mask_body, 0, unroll=4)

        def issue(k, p):
            off = k * CHUNK
            pltpu.async_copy(a_hbm.at[src_v.at[pl.ds(off, CHUNK)]],
                             rows_a[p], sem_a[p])
            pltpu.async_copy(b_hbm.at[dst_v.at[pl.ds(off, CHUNK)]],
                             rows_b[p], sem_b[p])

        def process(k, p):
            off = k * CHUNK
            pltpu.make_async_copy(a_hbm.at[pl.ds(0, CHUNK)],
                                  rows_a[p], sem_a[p]).wait()
            pltpu.make_async_copy(b_hbm.at[pl.ds(0, CHUNK)],
                                  rows_b[p], sem_b[p]).wait()

            @pl.when(k >= 2)
            def _():
                pltpu.make_async_copy(rows_o[p], out_hbm.at[pl.ds(0, CHUNK)],
                                      sem_w[p]).wait()

            def row_body(i, carry):
                for j in range(D // 16):
                    sl = pl.ds(j * 16, 16)
                    v = rows_a[p][i, sl] + rows_b[p][i, sl]
                    rows_o[p][i, sl] = jnp.maximum(v, v * 0.01)
                return carry

            lax.fori_loop(0, CHUNK, row_body, 0, unroll=4)
            pltpu.async_copy(rows_o[p], out_hbm.at[pl.ds(base + off, CHUNK)],
                             sem_w[p])

        issue(0, 0)

        def outer_body(i, carry):
            k0 = 2 * i
            issue(k0 + 1, 1)
            process(k0, 0)
            issue(k0 + 2, 0)
            process(k0 + 1, 1)
            return carry

        lax.fori_loop(0, (n_chunks - 1) // 2, outer_body, 0)
        process(n_chunks - 1, 0)

        pltpu.make_async_copy(rows_o[0], out_hbm.at[pl.ds(0, CHUNK)],
                              sem_w[0]).wait()
        pltpu.make_async_copy(rows_o[1], out_hbm.at[pl.ds(0, CHUNK)],
                              sem_w[1]).wait()

    return edge_kernel


def kernel(nf, edge_index, connected, W):
    ei = edge_index.astype(jnp.int32)
    src = ei[0]
    dst = ei[1]
    conn = connected.astype(jnp.int32)
    nf_pad = jnp.zeros((N_PAD, D), jnp.float32).at[:N, :].set(nf)
    w1 = W[:D, :]
    w2 = W[D:, :]
    a, b = _precompute_tables(nf_pad, w1, w2)

    info = plsc.get_sparse_core_info()
    num_workers = info.num_cores * info.num_subcores
    epw = E // num_workers
    edge_fn = _make_edge_kernel(num_workers, epw)
    return edge_fn(a, b, src, dst, conn)
